# baseline (speedup 1.0000x reference)
import numpy as np
import jax
import jax.numpy as jnp
from jax.experimental import pallas as pl

_KNEI = 64


def _fps(pos, n_sample):
    Np = pos.shape[0]

    def body(i, state):
        idx, dists, last = state
        d = jnp.sum((pos - pos[last]) ** 2, axis=-1)
        dists = jnp.minimum(dists, d)
        nxt = jnp.argmax(dists).astype(jnp.int32)
        return idx.at[i].set(nxt), dists, nxt

    idx0 = jnp.zeros((n_sample,), dtype=jnp.int32)
    d0 = jnp.full((Np,), jnp.inf, dtype=pos.dtype)
    idx, _, _ = jax.lax.fori_loop(1, n_sample, body, (idx0, d0, jnp.int32(0)))
    return idx


def _msg_kernel(xj_ref, valid_ref, w_ref, b_ref, o_ref):
    xv = xj_ref[...]
    r, k, cin = xv.shape
    m = jnp.dot(xv.reshape(r * k, cin), w_ref[...],
                preferred_element_type=jnp.float32) + b_ref[...]
    m = m.reshape(r, k, -1)
    m = jnp.where(valid_ref[...][..., None] > 0.0, m, -jnp.inf)
    o_ref[...] = jnp.max(m, axis=1)


def _msg_max(xj, valid, W, b, block_rows=64):
    M, K, Cin = xj.shape
    Cout = W.shape[1]
    Mp = int(np.ceil(M / block_rows)) * block_rows
    if Mp != M:
        xj = jnp.pad(xj, ((0, Mp - M), (0, 0), (0, 0)))
        valid = jnp.pad(valid, ((0, Mp - M), (0, 0)))
    out = pl.pallas_call(
        _msg_kernel,
        grid=(Mp // block_rows,),
        in_specs=[
            pl.BlockSpec((block_rows, K, Cin), lambda i: (i, 0, 0)),
            pl.BlockSpec((block_rows, K), lambda i: (i, 0)),
            pl.BlockSpec((Cin, Cout), lambda i: (0, 0)),
            pl.BlockSpec((1, Cout), lambda i: (0, 0)),
        ],
        out_specs=pl.BlockSpec((block_rows, Cout), lambda i: (i, 0)),
        out_shape=jax.ShapeDtypeStruct((Mp, Cout), jnp.float32),
    )(xj, valid, W, b.reshape(1, Cout))
    return out[:M]


def _linear_kernel(x_ref, w_ref, b_ref, o_ref):
    o_ref[...] = jnp.dot(x_ref[...], w_ref[...],
                         preferred_element_type=jnp.float32) + b_ref[...]


def _linear(x, W, b, block_rows=512):
    M, Cin = x.shape
    Cout = W.shape[1]
    Mp = int(np.ceil(M / block_rows)) * block_rows
    if Mp != M:
        x = jnp.pad(x, ((0, Mp - M), (0, 0)))
    out = pl.pallas_call(
        _linear_kernel,
        grid=(Mp // block_rows,),
        in_specs=[
            pl.BlockSpec((block_rows, Cin), lambda i: (i, 0)),
            pl.BlockSpec((Cin, Cout), lambda i: (0, 0)),
            pl.BlockSpec((1, Cout), lambda i: (0, 0)),
        ],
        out_specs=pl.BlockSpec((block_rows, Cout), lambda i: (i, 0)),
        out_shape=jax.ShapeDtypeStruct((Mp, Cout), jnp.float32),
    )(x, W, b.reshape(1, Cout))
    return out[:M]


def _gmax_kernel(x_ref, w_ref, b_ref, o_ref):
    g = jnp.dot(x_ref[0], w_ref[...],
                preferred_element_type=jnp.float32) + b_ref[...]
    o_ref[...] = jnp.max(g, axis=0).reshape(o_ref.shape)


def _global_max(x, W, b):
    Bb, S, Cin = x.shape
    Cout = W.shape[1]
    out = pl.pallas_call(
        _gmax_kernel,
        grid=(Bb,),
        in_specs=[
            pl.BlockSpec((1, S, Cin), lambda i: (i, 0, 0)),
            pl.BlockSpec((Cin, Cout), lambda i: (0, 0)),
            pl.BlockSpec((1, Cout), lambda i: (0, 0)),
        ],
        out_specs=pl.BlockSpec((1, 1, Cout), lambda i: (i, 0, 0)),
        out_shape=jax.ShapeDtypeStruct((Bb, 1, Cout), jnp.float32),
    )(x, W, b.reshape(1, Cout))
    return out


def _set_abstraction(feat, pos, ratio, r, W, b):
    Bb, Np = pos.shape[0], pos.shape[1]
    S = int(np.ceil(ratio * Np))
    idx = jax.vmap(lambda p: _fps(p, S))(pos)
    pos_dst = jnp.take_along_axis(pos, idx[..., None], axis=1)
    d2 = jnp.sum((pos_dst[:, :, None, :] - pos[:, None, :, :]) ** 2, axis=-1)
    within = d2 <= r * r
    K = min(_KNEI, Np)
    neg = jnp.where(within, -d2, -jnp.inf)
    vals, nbr = jax.lax.top_k(neg, K)
    valid = (vals > -jnp.inf).astype(jnp.float32)
    flat = nbr.reshape(Bb, -1)
    pos_j = jnp.take_along_axis(pos, flat[..., None], axis=1).reshape(
        Bb, S, K, pos.shape[-1])
    rel = pos_j - pos_dst[:, :, None, :]
    x_j = jnp.take_along_axis(feat, flat[..., None], axis=1).reshape(
        Bb, S, K, feat.shape[-1])
    msg_in = jnp.concatenate([x_j, rel], axis=-1)
    Cin = msg_in.shape[-1]
    out = _msg_max(msg_in.reshape(Bb * S, K, Cin), valid.reshape(Bb * S, K),
                   W, b)
    return out.reshape(Bb, S, -1), pos_dst


def _knn_interpolate(x, pos, pos_skip, k=8):
    Bb, Ns = pos_skip.shape[0], pos_skip.shape[1]
    d2 = jnp.sum((pos_skip[:, :, None, :] - pos[:, None, :, :]) ** 2, axis=-1)
    negd, nbr = jax.lax.top_k(-d2, k)
    w = 1.0 / jnp.maximum(-negd, 1e-16)
    x_j = jnp.take_along_axis(
        x, nbr.reshape(Bb, -1)[..., None], axis=1).reshape(Bb, Ns, k, x.shape[-1])
    return jnp.sum(x_j * w[..., None], axis=2) / jnp.sum(w, axis=2, keepdims=True)


def kernel(x, zones_ids, boundary_id, W0, b0, Wlf, blf, W1, b1, W2, b2, W3, b3):
    pos = x
    feats = jnp.concatenate([x, zones_ids, boundary_id], axis=2)
    Bb, Np = pos.shape[0], pos.shape[1]
    lx, lpos = _set_abstraction(feats, pos, 0.6, 0.2, W0, b0)
    ly = _knn_interpolate(lx, lpos, pos, k=8)
    lf_in = jnp.concatenate([ly, feats], axis=-1)
    local_features = _linear(
        lf_in.reshape(Bb * Np, lf_in.shape[-1]), Wlf, blf).reshape(Bb, Np, -1)
    g1, p1 = _set_abstraction(feats, pos, 0.5, 0.5, W1, b1)
    g2, p2 = _set_abstraction(g1, p1, 0.25, 0.8, W2, b2)
    g_in = jnp.concatenate([g2, p2], axis=-1)
    global_feature = _global_max(g_in, W3, b3)
    return local_features, global_feature


# speedup vs baseline: 1.2995x; 1.2995x over previous
import numpy as np
import jax
import jax.numpy as jnp
from jax.experimental import pallas as pl

_KNEI = 64


def _fps_kernel(px_ref, py_ref, o_ref):
    px = px_ref[...]
    py = py_ref[...]
    Bb, Np = px.shape
    S = o_ref.shape[0]
    iota = jax.lax.broadcasted_iota(jnp.int32, (Bb, Np), 1)
    o_ref[pl.ds(0, 1), :] = jnp.zeros((1, Bb), jnp.int32)

    def body(i, carry):
        dists, lastx, lasty = carry
        d = (px - lastx) ** 2 + (py - lasty) ** 2
        dists = jnp.minimum(dists, d)
        m = jnp.max(dists, axis=1, keepdims=True)
        cand = jnp.where(dists == m, iota, Np)
        nxt = jnp.min(cand, axis=1)
        o_ref[pl.ds(i, 1), :] = nxt.reshape(1, Bb)
        onehot = iota == nxt[:, None]
        lastx = jnp.sum(jnp.where(onehot, px, 0.0), axis=1, keepdims=True)
        lasty = jnp.sum(jnp.where(onehot, py, 0.0), axis=1, keepdims=True)
        return dists, lastx, lasty

    dists0 = jnp.full((Bb, Np), jnp.inf, jnp.float32)
    jax.lax.fori_loop(1, S, body, (dists0, px[:, 0:1], py[:, 0:1]))


def _fps_batched(pos, n_sample):
    Bb, Np = pos.shape[0], pos.shape[1]
    px = pos[..., 0]
    py = pos[..., 1]
    idx = pl.pallas_call(
        _fps_kernel,
        grid=(1,),
        in_specs=[
            pl.BlockSpec((Bb, Np), lambda i: (0, 0)),
            pl.BlockSpec((Bb, Np), lambda i: (0, 0)),
        ],
        out_specs=pl.BlockSpec((n_sample, Bb), lambda i: (0, 0)),
        out_shape=jax.ShapeDtypeStruct((n_sample, Bb), jnp.int32),
    )(px, py)
    return idx.T


def _msg_kernel(xj_ref, valid_ref, w_ref, b_ref, o_ref):
    xv = xj_ref[...]
    r, k, cin = xv.shape
    m = jnp.dot(xv.reshape(r * k, cin), w_ref[...],
                preferred_element_type=jnp.float32) + b_ref[...]
    m = m.reshape(r, k, -1)
    m = jnp.where(valid_ref[...][..., None] > 0.0, m, -jnp.inf)
    o_ref[...] = jnp.max(m, axis=1)


def _msg_max(xj, valid, W, b, block_rows=64):
    M, K, Cin = xj.shape
    Cout = W.shape[1]
    Mp = int(np.ceil(M / block_rows)) * block_rows
    if Mp != M:
        xj = jnp.pad(xj, ((0, Mp - M), (0, 0), (0, 0)))
        valid = jnp.pad(valid, ((0, Mp - M), (0, 0)))
    out = pl.pallas_call(
        _msg_kernel,
        grid=(Mp // block_rows,),
        in_specs=[
            pl.BlockSpec((block_rows, K, Cin), lambda i: (i, 0, 0)),
            pl.BlockSpec((block_rows, K), lambda i: (i, 0)),
            pl.BlockSpec((Cin, Cout), lambda i: (0, 0)),
            pl.BlockSpec((1, Cout), lambda i: (0, 0)),
        ],
        out_specs=pl.BlockSpec((block_rows, Cout), lambda i: (i, 0)),
        out_shape=jax.ShapeDtypeStruct((Mp, Cout), jnp.float32),
    )(xj, valid, W, b.reshape(1, Cout))
    return out[:M]


def _linear_kernel(x_ref, w_ref, b_ref, o_ref):
    o_ref[...] = jnp.dot(x_ref[...], w_ref[...],
                         preferred_element_type=jnp.float32) + b_ref[...]


def _linear(x, W, b, block_rows=512):
    M, Cin = x.shape
    Cout = W.shape[1]
    Mp = int(np.ceil(M / block_rows)) * block_rows
    if Mp != M:
        x = jnp.pad(x, ((0, Mp - M), (0, 0)))
    out = pl.pallas_call(
        _linear_kernel,
        grid=(Mp // block_rows,),
        in_specs=[
            pl.BlockSpec((block_rows, Cin), lambda i: (i, 0)),
            pl.BlockSpec((Cin, Cout), lambda i: (0, 0)),
            pl.BlockSpec((1, Cout), lambda i: (0, 0)),
        ],
        out_specs=pl.BlockSpec((block_rows, Cout), lambda i: (i, 0)),
        out_shape=jax.ShapeDtypeStruct((Mp, Cout), jnp.float32),
    )(x, W, b.reshape(1, Cout))
    return out[:M]


def _gmax_kernel(x_ref, w_ref, b_ref, o_ref):
    g = jnp.dot(x_ref[0], w_ref[...],
                preferred_element_type=jnp.float32) + b_ref[...]
    o_ref[...] = jnp.max(g, axis=0).reshape(o_ref.shape)


def _global_max(x, W, b):
    Bb, S, Cin = x.shape
    Cout = W.shape[1]
    out = pl.pallas_call(
        _gmax_kernel,
        grid=(Bb,),
        in_specs=[
            pl.BlockSpec((1, S, Cin), lambda i: (i, 0, 0)),
            pl.BlockSpec((Cin, Cout), lambda i: (0, 0)),
            pl.BlockSpec((1, Cout), lambda i: (0, 0)),
        ],
        out_specs=pl.BlockSpec((1, 1, Cout), lambda i: (i, 0, 0)),
        out_shape=jax.ShapeDtypeStruct((Bb, 1, Cout), jnp.float32),
    )(x, W, b.reshape(1, Cout))
    return out


def _set_abstraction(feat, pos, ratio, r, W, b):
    Bb, Np = pos.shape[0], pos.shape[1]
    S = int(np.ceil(ratio * Np))
    idx = _fps_batched(pos, S)
    pos_dst = jnp.take_along_axis(pos, idx[..., None], axis=1)
    d2 = jnp.sum((pos_dst[:, :, None, :] - pos[:, None, :, :]) ** 2, axis=-1)
    within = d2 <= r * r
    K = min(_KNEI, Np)
    neg = jnp.where(within, -d2, -jnp.inf)
    vals, nbr = jax.lax.top_k(neg, K)
    valid = (vals > -jnp.inf).astype(jnp.float32)
    flat = nbr.reshape(Bb, -1)
    pos_j = jnp.take_along_axis(pos, flat[..., None], axis=1).reshape(
        Bb, S, K, pos.shape[-1])
    rel = pos_j - pos_dst[:, :, None, :]
    x_j = jnp.take_along_axis(feat, flat[..., None], axis=1).reshape(
        Bb, S, K, feat.shape[-1])
    msg_in = jnp.concatenate([x_j, rel], axis=-1)
    Cin = msg_in.shape[-1]
    out = _msg_max(msg_in.reshape(Bb * S, K, Cin), valid.reshape(Bb * S, K),
                   W, b)
    return out.reshape(Bb, S, -1), pos_dst


def _knn_interpolate(x, pos, pos_skip, k=8):
    Bb, Ns = pos_skip.shape[0], pos_skip.shape[1]
    d2 = jnp.sum((pos_skip[:, :, None, :] - pos[:, None, :, :]) ** 2, axis=-1)
    negd, nbr = jax.lax.top_k(-d2, k)
    w = 1.0 / jnp.maximum(-negd, 1e-16)
    x_j = jnp.take_along_axis(
        x, nbr.reshape(Bb, -1)[..., None], axis=1).reshape(Bb, Ns, k, x.shape[-1])
    return jnp.sum(x_j * w[..., None], axis=2) / jnp.sum(w, axis=2, keepdims=True)


def kernel(x, zones_ids, boundary_id, W0, b0, Wlf, blf, W1, b1, W2, b2, W3, b3):
    pos = x
    feats = jnp.concatenate([x, zones_ids, boundary_id], axis=2)
    Bb, Np = pos.shape[0], pos.shape[1]
    lx, lpos = _set_abstraction(feats, pos, 0.6, 0.2, W0, b0)
    ly = _knn_interpolate(lx, lpos, pos, k=8)
    lf_in = jnp.concatenate([ly, feats], axis=-1)
    local_features = _linear(
        lf_in.reshape(Bb * Np, lf_in.shape[-1]), Wlf, blf).reshape(Bb, Np, -1)
    g1, p1 = _set_abstraction(feats, pos, 0.5, 0.5, W1, b1)
    g2, p2 = _set_abstraction(g1, p1, 0.25, 0.8, W2, b2)
    g_in = jnp.concatenate([g2, p2], axis=-1)
    global_feature = _global_max(g_in, W3, b3)
    return local_features, global_feature
